# Initial kernel scaffold; baseline (speedup 1.0000x reference)
import jax
import jax.numpy as jnp
from jax.experimental import pallas as pl

_RADIUS2 = 1.0
_RB = 256


def _bq_kernel(xyz_ref, xyzt_ref, out_ref):
    n = xyzt_ref.shape[2]
    xr = xyz_ref[0]
    x0r = xr[:, 0][:, None]
    x1r = xr[:, 1][:, None]
    x2r = xr[:, 2][:, None]
    xc = xyzt_ref[0]
    x0c = xc[0, :][None, :]
    x1c = xc[1, :][None, :]
    x2c = xc[2, :][None, :]
    sq_r = x0r * x0r + x1r * x1r + x2r * x2r
    sq_c = x0c * x0c + x1c * x1c + x2c * x2c
    dot = x0r * x0c + x1r * x1c + x2r * x2c
    d2 = sq_r + sq_c - 2.0 * dot
    mask = d2 < _RADIUS2
    col = jax.lax.broadcasted_iota(jnp.int32, mask.shape, 1)
    enc = jnp.where(mask, col, n)
    first = jnp.min(enc, axis=1)
    first = jnp.where(first == n, 0, first)
    out_ref[0] = first[:, None]


def kernel(p):
    b, n, _ = p.shape
    xyz = p[:, :, 0:3]
    xyzt = jnp.transpose(xyz, (0, 2, 1))
    out = pl.pallas_call(
        _bq_kernel,
        grid=(b, n // _RB),
        in_specs=[
            pl.BlockSpec((1, _RB, 3), lambda bi, r: (bi, r, 0)),
            pl.BlockSpec((1, 3, n), lambda bi, r: (bi, 0, 0)),
        ],
        out_specs=pl.BlockSpec((1, _RB, 1), lambda bi, r: (bi, r, 0)),
        out_shape=jax.ShapeDtypeStruct((b, n, 1), jnp.int32),
    )(xyz, xyzt)
    return out


# speedup vs baseline: 1.1686x; 1.1686x over previous
import jax
import jax.numpy as jnp
from jax.experimental import pallas as pl

_RADIUS2 = 1.0
_RB = 256


def _bq_kernel(xyz_ref, xyzt_ref, out_ref):
    n = xyzt_ref.shape[2]
    xr = xyz_ref[0]
    x0r = xr[:, 0][:, None]
    x1r = xr[:, 1][:, None]
    x2r = xr[:, 2][:, None]
    xc = xyzt_ref[0]
    x0c = xc[0, :][None, :]
    x1c = xc[1, :][None, :]
    x2c = xc[2, :][None, :]
    sq_r = x0r * x0r + x1r * x1r + x2r * x2r
    sq_c = x0c * x0c + x1c * x1c + x2c * x2c
    b = lambda v: v.astype(jnp.bfloat16).astype(jnp.float32)
    dot = b(x0r) * b(x0c) + b(x1r) * b(x1c) + b(x2r) * b(x2c)
    d2 = sq_r + sq_c - 2.0 * dot
    mask = d2 < _RADIUS2
    col = jax.lax.broadcasted_iota(jnp.int32, mask.shape, 1)
    enc = jnp.where(mask, col, n)
    first = jnp.min(enc, axis=1)
    first = jnp.where(first == n, 0, first)
    out_ref[0] = first[:, None]


def kernel(p):
    b, n, _ = p.shape
    xyz = p[:, :, 0:3]
    xyzt = jnp.transpose(xyz, (0, 2, 1))
    out = pl.pallas_call(
        _bq_kernel,
        grid=(b, n // _RB),
        in_specs=[
            pl.BlockSpec((1, _RB, 3), lambda bi, r: (bi, r, 0)),
            pl.BlockSpec((1, 3, n), lambda bi, r: (bi, 0, 0)),
        ],
        out_specs=pl.BlockSpec((1, _RB, 1), lambda bi, r: (bi, r, 0)),
        out_shape=jax.ShapeDtypeStruct((b, n, 1), jnp.int32),
    )(xyz, xyzt)
    return out
